# Initial kernel scaffold; baseline (speedup 1.0000x reference)
import jax
import jax.numpy as jnp
from jax.experimental import pallas as pl


def kernel(x, edge_attr, W1, b1, We, gfeat, Wc, bc, edge_index, batch):
    raise NotImplementedError("write your pallas kernel here")


# speedup vs baseline: 2.5256x; 2.5256x over previous
import functools

import jax
import jax.numpy as jnp
from jax import lax
from jax.experimental import pallas as pl
from jax.experimental.pallas import tpu as pltpu
from jax.experimental.pallas import tpu_sc as plsc

N = 10000
E = 320000
D = 128
DE = 4
DG = 32
NG = 64
NC = 8

NW = 32
CHUNK = 128
G = 80
EP = NW * G * CHUNK
RB = 1000
NBLK = N // RB
NP_ = 10240


def _phase_a_body(x_ref, ea_ref, w1_ref, b1_ref, s_ref, h_ref, wr_ref):
    h_ref[...] = (
        jnp.dot(x_ref[...], w1_ref[...], preferred_element_type=jnp.float32)
        + b1_ref[...]
    )
    e = jnp.exp(-(ea_ref[...] * ea_ref[...]))
    wr_ref[...] = jnp.dot(e, s_ref[...], preferred_element_type=jnp.float32)


def _phase_a(x, ea_r, W1, b1, S):
    return pl.pallas_call(
        _phase_a_body,
        grid=(NBLK,),
        in_specs=[
            pl.BlockSpec((RB, D), lambda i: (i, 0)),
            pl.BlockSpec((RB, D), lambda i: (i, 0)),
            pl.BlockSpec((D, D), lambda i: (0, 0)),
            pl.BlockSpec((1, D), lambda i: (0, 0)),
            pl.BlockSpec((D, 32), lambda i: (0, 0)),
        ],
        out_specs=[
            pl.BlockSpec((RB, D), lambda i: (i, 0)),
            pl.BlockSpec((RB, 32), lambda i: (i, 0)),
        ],
        out_shape=[
            jax.ShapeDtypeStruct((N, D), jnp.float32),
            jax.ShapeDtypeStruct((N, 32), jnp.float32),
        ],
    )(x, ea_r, W1, b1, S)


def _phase_b_body(h_hbm, src_hbm, dst_hbm, w_hbm, zeros_hbm, out_hbm,
                  src_v, dst_v, w_v, rows, agg_sh):
    cid = lax.axis_index("c")
    sid = lax.axis_index("s")
    wid = sid * 2 + cid
    rows_per_tile = NP_ // 16

    pltpu.sync_copy(zeros_hbm.at[pl.ds(sid * rows_per_tile, rows_per_tile)],
                    agg_sh.at[pl.ds(sid * rows_per_tile, rows_per_tile)])
    pltpu.sync_copy(src_hbm.at[wid], src_v)
    pltpu.sync_copy(dst_hbm.at[wid], dst_v)
    pltpu.sync_copy(w_hbm.at[wid], w_v)
    plsc.subcore_barrier()

    @pl.loop(0, G)
    def _chunk(g):
        pltpu.sync_copy(h_hbm.at[src_v.at[g]], rows)

        @pl.loop(0, CHUNK // 16)
        def _scale(jo):
            wv = w_v[g, pl.ds(jo * 16, 16)]
            for ji in range(16):
                j = jo * 16 + ji
                wj = wv[ji]
                for k in range(D // 16):
                    sl = pl.ds(k * 16, 16)
                    rows[j, sl] = rows[j, sl] * wj

        pltpu.sync_copy(rows, agg_sh.at[dst_v.at[g]], add=True)

    plsc.subcore_barrier()
    pltpu.sync_copy(agg_sh.at[pl.ds(sid * rows_per_tile, rows_per_tile)],
                    out_hbm.at[cid, pl.ds(sid * rows_per_tile, rows_per_tile)])


def _phase_b(h, src_r, dst_r, w_r, zeros):
    mesh = plsc.VectorSubcoreMesh(core_axis_name="c", subcore_axis_name="s")
    kern = pl.kernel(
        _phase_b_body,
        out_type=jax.ShapeDtypeStruct((2, NP_, D), jnp.float32),
        mesh=mesh,
        scratch_types=[
            pltpu.VMEM((G, CHUNK), jnp.int32),
            pltpu.VMEM((G, CHUNK), jnp.int32),
            pltpu.VMEM((G, CHUNK), jnp.float32),
            pltpu.VMEM((CHUNK, D), jnp.float32),
            pltpu.VMEM_SHARED((NP_, D), jnp.float32),
        ],
    )
    return kern(h, src_r, dst_r, w_r, zeros)


def _phase_c_body(p_ref, h_ref, batch_ref, wct_ref, gf_ref, wcb_ref, bc_ref,
                  out_ref, pooled):
    i = pl.program_id(0)

    @pl.when(i == 0)
    def _():
        pooled[...] = jnp.zeros((NG, D), jnp.float32)

    emb = jax.nn.relu(p_ref[0] + p_ref[1] + h_ref[...])
    b = batch_ref[0, 0, :]
    iota = lax.broadcasted_iota(jnp.int32, (NG, RB), 0)
    onehot_t = (iota == b[None, :]).astype(jnp.float32)
    pooled[...] += jnp.dot(onehot_t, emb, preferred_element_type=jnp.float32)

    @pl.when(i == NBLK - 1)
    def _():
        out_ref[...] = (
            jnp.dot(pooled[...], wct_ref[...],
                    preferred_element_type=jnp.float32)
            + jnp.dot(gf_ref[...], wcb_ref[...],
                      preferred_element_type=jnp.float32)
            + bc_ref[...]
        )


def _phase_c(partials, h, batch3, Wc_top, gfeat, Wc_bot, bc2):
    return pl.pallas_call(
        _phase_c_body,
        grid=(NBLK,),
        in_specs=[
            pl.BlockSpec((2, RB, D), lambda i: (0, i, 0)),
            pl.BlockSpec((RB, D), lambda i: (i, 0)),
            pl.BlockSpec((1, 1, RB), lambda i: (i, 0, 0)),
            pl.BlockSpec((D, NC), lambda i: (0, 0)),
            pl.BlockSpec((1, DG), lambda i: (0, 0)),
            pl.BlockSpec((DG, NC), lambda i: (0, 0)),
            pl.BlockSpec((1, NC), lambda i: (0, 0)),
        ],
        out_specs=pl.BlockSpec((NG, NC), lambda i: (0, 0)),
        out_shape=jax.ShapeDtypeStruct((NG, NC), jnp.float32),
        scratch_shapes=[pltpu.VMEM((NG, D), jnp.float32)],
    )(partials, h, batch3, Wc_top, gfeat, Wc_bot, bc2)


@jax.jit
def kernel(x, edge_attr, W1, b1, We, gfeat, Wc, bc, edge_index, batch):
    r = jnp.arange(D)
    S = jnp.zeros((D, 32), jnp.float32).at[r, r // DE].set(We[r % DE, 0])
    ea_r = edge_attr.reshape(N, D)

    h, wr = _phase_a(x, ea_r, W1, b1.reshape(1, D), S)

    pad = EP - E
    src = jnp.concatenate(
        [edge_index[0].astype(jnp.int32), jnp.zeros((pad,), jnp.int32)]
    ).reshape(NW, G, CHUNK)
    dst = jnp.concatenate(
        [edge_index[1].astype(jnp.int32), jnp.zeros((pad,), jnp.int32)]
    ).reshape(NW, G, CHUNK)
    w_flat = jnp.concatenate([wr.reshape(-1), jnp.zeros((pad,), jnp.float32)])
    w_r = w_flat.reshape(NW, G, CHUNK)
    zeros = jnp.zeros((NP_, D), jnp.float32)

    partials = _phase_b(h, src, dst, w_r, zeros)[:, :N, :]

    batch3 = batch.astype(jnp.int32).reshape(NBLK, 1, RB)
    return _phase_c(partials, h, batch3, Wc[:D], gfeat, Wc[D:], bc.reshape(1, NC))


# speedup vs baseline: 2.8844x; 1.1421x over previous
import functools

import jax
import jax.numpy as jnp
from jax import lax
from jax.experimental import pallas as pl
from jax.experimental.pallas import tpu as pltpu
from jax.experimental.pallas import tpu_sc as plsc

N = 10000
E = 320000
D = 128
DE = 4
DG = 32
NG = 64
NC = 8

NW = 32
CHUNK = 64
IDXG = 32
CG = 5
G = CG * IDXG
EP = NW * G * CHUNK
RB = 1000
NBLK = N // RB
NP_ = 10240


def _phase_a_body(x_ref, ea_ref, w1_ref, b1_ref, s_ref, h_ref, wr_ref):
    h_ref[...] = (
        jnp.dot(x_ref[...], w1_ref[...], preferred_element_type=jnp.float32)
        + b1_ref[...]
    )
    e = jnp.exp(-(ea_ref[...] * ea_ref[...]))
    wr_ref[...] = jnp.dot(e, s_ref[...], preferred_element_type=jnp.float32)


def _phase_a(x, ea_r, W1, b1, S):
    return pl.pallas_call(
        _phase_a_body,
        grid=(NBLK,),
        in_specs=[
            pl.BlockSpec((RB, D), lambda i: (i, 0)),
            pl.BlockSpec((RB, D), lambda i: (i, 0)),
            pl.BlockSpec((D, D), lambda i: (0, 0)),
            pl.BlockSpec((1, D), lambda i: (0, 0)),
            pl.BlockSpec((D, 32), lambda i: (0, 0)),
        ],
        out_specs=[
            pl.BlockSpec((RB, D), lambda i: (i, 0)),
            pl.BlockSpec((RB, 32), lambda i: (i, 0)),
        ],
        out_shape=[
            jax.ShapeDtypeStruct((N, D), jnp.float32),
            jax.ShapeDtypeStruct((N, 32), jnp.float32),
        ],
    )(x, ea_r, W1, b1, S)


NBUF = 4
PREF = 2


def _phase_b_body(h_hbm, src_hbm, dst_hbm, w_hbm, zeros_hbm, out_hbm,
                  src_v, dst_v, w_v, b0, b1, b2, b3, agg_sh,
                  g0s, g1s, g2s, g3s, s0s, s1s, s2s, s3s):
    cid = lax.axis_index("c")
    sid = lax.axis_index("s")
    wid = sid * 2 + cid
    rows_per_tile = NP_ // 16
    bufs = [b0, b1, b2, b3]
    gsem = [g0s, g1s, g2s, g3s]
    ssem = [s0s, s1s, s2s, s3s]

    def gather_start(j, b):
        pltpu.async_copy(h_hbm.at[src_v.at[j]], bufs[b], gsem[b])

    def gather_wait(j, b):
        pltpu.make_async_copy(h_hbm.at[src_v.at[j]], bufs[b], gsem[b]).wait()

    def scatter_start(j, b):
        pltpu.async_copy(bufs[b], agg_sh.at[dst_v.at[j]], ssem[b], add=True)

    def scatter_wait(j, b):
        pltpu.make_async_copy(bufs[b], agg_sh.at[dst_v.at[j]], ssem[b]).wait()

    def scale(j, b):
        @pl.loop(0, CHUNK // 16)
        def _scale(jo):
            wv = w_v[j, pl.ds(jo * 16, 16)]
            for ji in range(16):
                jr = jo * 16 + ji
                wj = wv[ji]
                for k in range(D // 16):
                    sl = pl.ds(k * 16, 16)
                    bufs[b][jr, sl] = bufs[b][jr, sl] * wj

    pltpu.sync_copy(zeros_hbm.at[pl.ds(sid * rows_per_tile, rows_per_tile)],
                    agg_sh.at[pl.ds(sid * rows_per_tile, rows_per_tile)])
    plsc.subcore_barrier()

    @pl.loop(0, CG)
    def _staging_group(cg):
        @pl.when(cg > 0)
        def _():
            for bb in range(NBUF):
                scatter_wait(IDXG - NBUF + bb, bb)

        pltpu.sync_copy(src_hbm.at[wid, cg], src_v)
        pltpu.sync_copy(dst_hbm.at[wid, cg], dst_v)
        pltpu.sync_copy(w_hbm.at[wid, cg], w_v)

        for j in range(PREF):
            gather_start(j, j)

        @pl.loop(0, IDXG // NBUF)
        def _group(grp):
            for bb in range(NBUF):
                j = grp * NBUF + bb
                bp = (bb + PREF) % NBUF
                jp = j + PREF
                if bb < PREF:
                    @pl.when(grp > 0)
                    def _():
                        scatter_wait(jp - NBUF, bp)
                        gather_start(jp, bp)

                    @pl.when(grp == 0)
                    def _():
                        gather_start(jp, bp)
                else:
                    @pl.when(grp < IDXG // NBUF - 1)
                    def _():
                        scatter_wait(jp - NBUF, bp)
                        gather_start(jp, bp)
                gather_wait(j, bb)
                scale(j, bb)
                scatter_start(j, bb)

    for bb in range(NBUF):
        scatter_wait(IDXG - NBUF + bb, bb)

    plsc.subcore_barrier()
    pltpu.sync_copy(agg_sh.at[pl.ds(sid * rows_per_tile, rows_per_tile)],
                    out_hbm.at[cid, pl.ds(sid * rows_per_tile, rows_per_tile)])


def _phase_b(h, src_r, dst_r, w_r, zeros):
    mesh = plsc.VectorSubcoreMesh(core_axis_name="c", subcore_axis_name="s")
    kern = pl.kernel(
        _phase_b_body,
        out_type=jax.ShapeDtypeStruct((2, NP_, D), jnp.float32),
        mesh=mesh,
        scratch_types=(
            [
                pltpu.VMEM((IDXG, CHUNK), jnp.int32),
                pltpu.VMEM((IDXG, CHUNK), jnp.int32),
                pltpu.VMEM((IDXG, CHUNK), jnp.float32),
            ]
            + [pltpu.VMEM((CHUNK, D), jnp.float32)] * NBUF
            + [pltpu.VMEM_SHARED((NP_, D), jnp.float32)]
            + [pltpu.SemaphoreType.DMA] * (2 * NBUF)
        ),
    )
    return kern(h, src_r, dst_r, w_r, zeros)


def _phase_c_body(p_ref, h_ref, batch_ref, wct_ref, gf_ref, wcb_ref, bc_ref,
                  out_ref, pooled):
    i = pl.program_id(0)

    @pl.when(i == 0)
    def _():
        pooled[...] = jnp.zeros((NG, D), jnp.float32)

    emb = jax.nn.relu(p_ref[0] + p_ref[1] + h_ref[...])
    b = batch_ref[0, 0, :]
    iota = lax.broadcasted_iota(jnp.int32, (NG, RB), 0)
    onehot_t = (iota == b[None, :]).astype(jnp.float32)
    pooled[...] += jnp.dot(onehot_t, emb, preferred_element_type=jnp.float32)

    @pl.when(i == NBLK - 1)
    def _():
        out_ref[...] = (
            jnp.dot(pooled[...], wct_ref[...],
                    preferred_element_type=jnp.float32)
            + jnp.dot(gf_ref[...], wcb_ref[...],
                      preferred_element_type=jnp.float32)
            + bc_ref[...]
        )


def _phase_c(partials, h, batch3, Wc_top, gfeat, Wc_bot, bc2):
    return pl.pallas_call(
        _phase_c_body,
        grid=(NBLK,),
        in_specs=[
            pl.BlockSpec((2, RB, D), lambda i: (0, i, 0)),
            pl.BlockSpec((RB, D), lambda i: (i, 0)),
            pl.BlockSpec((1, 1, RB), lambda i: (i, 0, 0)),
            pl.BlockSpec((D, NC), lambda i: (0, 0)),
            pl.BlockSpec((1, DG), lambda i: (0, 0)),
            pl.BlockSpec((DG, NC), lambda i: (0, 0)),
            pl.BlockSpec((1, NC), lambda i: (0, 0)),
        ],
        out_specs=pl.BlockSpec((NG, NC), lambda i: (0, 0)),
        out_shape=jax.ShapeDtypeStruct((NG, NC), jnp.float32),
        scratch_shapes=[pltpu.VMEM((NG, D), jnp.float32)],
    )(partials, h, batch3, Wc_top, gfeat, Wc_bot, bc2)


@jax.jit
def kernel(x, edge_attr, W1, b1, We, gfeat, Wc, bc, edge_index, batch):
    r = jnp.arange(D)
    S = jnp.zeros((D, 32), jnp.float32).at[r, r // DE].set(We[r % DE, 0])
    ea_r = edge_attr.reshape(N, D)

    h, wr = _phase_a(x, ea_r, W1, b1.reshape(1, D), S)

    pad = EP - E
    src = jnp.concatenate(
        [edge_index[0].astype(jnp.int32), jnp.zeros((pad,), jnp.int32)]
    ).reshape(NW, CG, IDXG, CHUNK)
    dst = jnp.concatenate(
        [edge_index[1].astype(jnp.int32), jnp.zeros((pad,), jnp.int32)]
    ).reshape(NW, CG, IDXG, CHUNK)
    w_flat = jnp.concatenate([wr.reshape(-1), jnp.zeros((pad,), jnp.float32)])
    w_r = w_flat.reshape(NW, CG, IDXG, CHUNK)
    zeros = jnp.zeros((NP_, D), jnp.float32)

    partials = _phase_b(h, src, dst, w_r, zeros)[:, :N, :]

    batch3 = batch.astype(jnp.int32).reshape(NBLK, 1, RB)
    return _phase_c(partials, h, batch3, Wc[:D], gfeat, Wc[D:], bc.reshape(1, NC))
